# Initial kernel scaffold; baseline (speedup 1.0000x reference)
import jax
import jax.numpy as jnp
from jax.experimental import pallas as pl


def kernel(vectors, subcategory, category, subcategory_repr, subcategory_weights, category_repr, category_weights, user_repr):
    raise NotImplementedError("write your pallas kernel here")


# speedup vs baseline: 176.4062x; 176.4062x over previous
import functools

import jax
import jax.numpy as jnp
from jax import lax
from jax.experimental import pallas as pl
from jax.experimental.pallas import tpu as pltpu
from jax.experimental.pallas import tpu_sc as plsc

B, K, D, NS, NC = 4096, 50, 64, 256, 18
LAM_T, LAM_S = 0.15, 0.7
LAM_U = 1.0 - LAM_S - LAM_T
L = 16
NWORK = 32
BPW = B // NWORK
CB = 4
NCHUNK = BPW // CB
KP = 64
KG = 4
UNROLL = 16


def _body(v_h, sip_h, ci_h, subr_h, sw_h, ctr_h, cw_h, u_h, out_h,
          v_l, es_l, si_l, ci_l, ct_l, sw_l, cw_l, u_l, out_l, sem):
    wid = lax.axis_index("s") * 2 + lax.axis_index("c")
    b0w = wid * BPW
    iota = lax.iota(jnp.int32, L)
    ci_l[pl.ds(CB * K, L)] = jnp.zeros((L,), jnp.int32)

    def chunk(c, carry):
        b0 = b0w + c * CB
        cps = [
            pltpu.async_copy(v_h.at[pl.ds(b0 * K, CB * K)],
                             v_l.at[pl.ds(0, CB * K)], sem),
            pltpu.async_copy(sip_h.at[pl.ds(b0, CB)], si_l, sem),
            pltpu.async_copy(ci_h.at[pl.ds(b0 * K, CB * K)],
                             ci_l.at[pl.ds(0, CB * K)], sem),
            pltpu.async_copy(ctr_h.at[pl.ds(b0 * NC, CB * NC)], ct_l, sem),
            pltpu.async_copy(sw_h.at[pl.ds(b0 * NS, CB * NS)], sw_l, sem),
            pltpu.async_copy(cw_h.at[pl.ds(b0 * NC, CB * NC)], cw_l, sem),
            pltpu.async_copy(u_h.at[pl.ds(b0, CB)], u_l, sem),
        ]
        for cp in cps:
            cp.wait()
        gcs = [
            pltpu.async_copy(subr_h.at[b0 + j].at[si_l.at[j]],
                             es_l.at[pl.ds(j * KP, KP)], sem)
            for j in range(CB)
        ]
        for cp in gcs:
            cp.wait()

        for j in range(CB):
            for g in range(KG):
                kvec = iota + g * L
                rv = kvec + j * K
                re = kvec + j * KP
                civ = plsc.load_gather(ci_l, [rv])
                rc = civ + j * NC

                def dstep(it, acc, rv=rv, re=re, rc=rc, j=j):
                    accs, accc, accu, cd = acc
                    uvec = u_l[j, pl.ds(it * UNROLL, UNROLL)]
                    for dd in range(UNROLL):
                        vg = plsc.load_gather(v_l, [rv, cd])
                        eg = plsc.load_gather(es_l, [re, cd])
                        cg = plsc.load_gather(ct_l, [rc, cd])
                        us = uvec[dd]
                        accs = accs + vg * eg
                        accc = accc + vg * cg
                        accu = accu + vg * us
                        cd = cd + 1
                    return accs, accc, accu, cd

                zf = jnp.zeros((L,), jnp.float32)
                zi = jnp.zeros((L,), jnp.int32)
                accs, accc, accu, _ = lax.fori_loop(
                    0, D // UNROLL, dstep, (zf, zf, zf, zi))

                siv = plsc.load_gather(si_l, [jnp.full((L,), j, jnp.int32),
                                              kvec])
                ws = plsc.load_gather(sw_l, [siv + j * NS])
                wc = plsc.load_gather(cw_l, [civ + j * NC])
                outv = LAM_S * ws * accs + LAM_T * wc * accc + LAM_U * accu
                plsc.store_scatter(out_l, [rv], outv, mask=kvec < K)

        pltpu.sync_copy(out_l.at[pl.ds(0, CB * K)],
                        out_h.at[pl.ds(b0 * K, CB * K)])
        return carry

    lax.fori_loop(0, NCHUNK, chunk, 0)


@jax.jit
def _run(v_f, si_p, ci_f, subr, sw_f, ctr_f, cw_f, user):
    mesh = plsc.VectorSubcoreMesh(core_axis_name="c", subcore_axis_name="s")
    f = pl.kernel(
        _body,
        out_type=jax.ShapeDtypeStruct((B * K,), jnp.float32),
        mesh=mesh,
        compiler_params=pltpu.CompilerParams(needs_layout_passes=False,
                                             use_tc_tiling_on_sc=False),
        scratch_types=[
            pltpu.VMEM((CB * K + L, D), jnp.float32),
            pltpu.VMEM((CB * KP, D), jnp.float32),
            pltpu.VMEM((CB, KP), jnp.int32),
            pltpu.VMEM((CB * K + L,), jnp.int32),
            pltpu.VMEM((CB * NC, D), jnp.float32),
            pltpu.VMEM((CB * NS,), jnp.float32),
            pltpu.VMEM((CB * NC,), jnp.float32),
            pltpu.VMEM((CB, D), jnp.float32),
            pltpu.VMEM((CB * K + L,), jnp.float32),
            pltpu.SemaphoreType.DMA,
        ],
    )
    return f(v_f, si_p, ci_f, subr, sw_f, ctr_f, cw_f, user)


def kernel(vectors, subcategory, category, subcategory_repr,
           subcategory_weights, category_repr, category_weights, user_repr):
    si_p = jnp.pad(subcategory.astype(jnp.int32), ((0, 0), (0, KP - K)))
    v_f = vectors.reshape(B * K, D)
    ci_f = category.astype(jnp.int32).reshape(B * K)
    ctr_f = category_repr.reshape(B * NC, D)
    sw_f = subcategory_weights.reshape(B * NS)
    cw_f = category_weights.reshape(B * NC)
    out = _run(v_f, si_p, ci_f, subcategory_repr, sw_f, ctr_f, cw_f,
               user_repr)
    return out.reshape(B, K)


# speedup vs baseline: 191.3712x; 1.0848x over previous
import jax
import jax.numpy as jnp
from jax import lax
from jax.experimental import pallas as pl
from jax.experimental.pallas import tpu as pltpu
from jax.experimental.pallas import tpu_sc as plsc

B, K, D, NS, NC = 4096, 50, 64, 256, 18
LAM_T, LAM_S = 0.15, 0.7
LAM_U = 1.0 - LAM_S - LAM_T
L = 16
NWORK = 32
BPW = B // NWORK
CB = 4
NCHUNK = BPW // CB
KP = 64
KG = 4
UNROLL = 16


def _lin_descs(hb, s, b0, sem):
    v_h, ci_h, ctr_h, sw_h, cw_h, u_h, sic_h = hb
    v_l, es_l, si_l, ci_l, ct_l, sw_l, cw_l, u_l, out_l, si_c = s
    return [
        (v_h.at[pl.ds(b0 * K, CB * K)], v_l.at[pl.ds(0, CB * K)], sem),
        (sic_h.at[pl.ds(b0 * K, CB * K)], si_c.at[pl.ds(0, CB * K)], sem),
        (ci_h.at[pl.ds(b0 * K, CB * K)], ci_l.at[pl.ds(0, CB * K)], sem),
        (ctr_h.at[pl.ds(b0 * NC, CB * NC)], ct_l, sem),
        (sw_h.at[pl.ds(b0 * NS, CB * NS)], sw_l, sem),
        (cw_h.at[pl.ds(b0 * NC, CB * NC)], cw_l, sem),
        (u_h.at[pl.ds(b0, CB)], u_l, sem),
    ]


def _gather_descs(subr_h, s, b0, sem):
    v_l, es_l, si_l, ci_l, ct_l, sw_l, cw_l, u_l, out_l, si_c = s
    return [
        (subr_h.at[b0 + j].at[si_l.at[j]], es_l.at[pl.ds(j * KP, KP)], sem)
        for j in range(CB)
    ]


def _compute(s, iota):
    v_l, es_l, si_l, ci_l, ct_l, sw_l, cw_l, u_l, out_l, si_c = s
    for j in range(CB):
        for g in range(KG):
            kvec = iota + g * L
            rv = kvec + j * K
            re = kvec + j * KP
            civ = plsc.load_gather(ci_l, [rv])
            rc = civ + j * NC

            def dstep(it, acc, rv=rv, re=re, rc=rc, j=j):
                accs, accc, accu, cd = acc
                uvec = u_l[j, pl.ds(it * UNROLL, UNROLL)]
                for dd in range(UNROLL):
                    vg = plsc.load_gather(v_l, [rv, cd])
                    eg = plsc.load_gather(es_l, [re, cd])
                    cg = plsc.load_gather(ct_l, [rc, cd])
                    us = uvec[dd]
                    accs = accs + vg * eg
                    accc = accc + vg * cg
                    accu = accu + vg * us
                    cd = cd + 1
                return accs, accc, accu, cd

            zf = jnp.zeros((L,), jnp.float32)
            zi = jnp.zeros((L,), jnp.int32)
            accs, accc, accu, _ = lax.fori_loop(
                0, D // UNROLL, dstep, (zf, zf, zf, zi))

            siv = plsc.load_gather(si_c, [rv])
            ws = plsc.load_gather(sw_l, [siv + j * NS])
            wc = plsc.load_gather(cw_l, [civ + j * NC])
            outv = LAM_S * ws * accs + LAM_T * wc * accc + LAM_U * accu
            plsc.store_scatter(out_l, [rv], outv, mask=kvec < K)


def _body(v_h, sip_h, sic_h, ci_h, subr_h, sw_h, ctr_h, cw_h, u_h, out_h,
          *sc):
    sets = [sc[0:10], sc[10:20]]
    sem_l = sc[20:22]
    sem_g = sc[22:24]
    sem_s = sc[24:26]
    sem_o = sc[26:28]
    hb = (v_h, ci_h, ctr_h, sw_h, cw_h, u_h, sic_h)
    wid = lax.axis_index("s") * 2 + lax.axis_index("c")
    b0w = wid * BPW
    iota = lax.iota(jnp.int32, L)
    for s in sets:
        s[3][pl.ds(CB * K, L)] = jnp.zeros((L,), jnp.int32)
        s[9][pl.ds(CB * K, L)] = jnp.zeros((L,), jnp.int32)

    def si_copy(b0, j2):
        return (sip_h.at[pl.ds(b0, CB)], sets[j2][2], sem_s[j2])

    pltpu.async_copy(*si_copy(b0w, 0))
    pltpu.async_copy(*si_copy(b0w + CB, 1))
    pltpu.make_async_copy(*si_copy(b0w, 0)).wait()
    for d in _lin_descs(hb, sets[0], b0w, sem_l[0]):
        pltpu.async_copy(*d)
    for d in _gather_descs(subr_h, sets[0], b0w, sem_g[0]):
        pltpu.async_copy(*d)

    def step(sstep, carry):
        for j2 in range(2):
            c = 2 * sstep + j2
            b0 = b0w + c * CB
            s, o = sets[j2], sets[1 - j2]
            for d in _lin_descs(hb, s, b0, sem_l[j2]):
                pltpu.make_async_copy(*d).wait()
            for d in _gather_descs(subr_h, s, b0, sem_g[j2]):
                pltpu.make_async_copy(*d).wait()

            @pl.when(c + 1 < NCHUNK)
            def _():
                bn = b0 + CB
                pltpu.make_async_copy(*si_copy(bn, 1 - j2)).wait()
                for d in _lin_descs(hb, o, bn, sem_l[1 - j2]):
                    pltpu.async_copy(*d)
                for d in _gather_descs(subr_h, o, bn, sem_g[1 - j2]):
                    pltpu.async_copy(*d)

            @pl.when(c + 2 < NCHUNK)
            def _():
                pltpu.async_copy(*si_copy(b0 + 2 * CB, j2))

            @pl.when(c >= 2)
            def _():
                pltpu.make_async_copy(
                    s[8].at[pl.ds(0, CB * K)],
                    out_h.at[pl.ds(b0w * K, CB * K)], sem_o[j2]).wait()

            _compute(s, iota)
            pltpu.async_copy(s[8].at[pl.ds(0, CB * K)],
                             out_h.at[pl.ds(b0 * K, CB * K)], sem_o[j2])
        return carry

    lax.fori_loop(0, NCHUNK // 2, step, 0)
    for j2 in range(2):
        pltpu.make_async_copy(sets[j2][8].at[pl.ds(0, CB * K)],
                              out_h.at[pl.ds(b0w * K, CB * K)],
                              sem_o[j2]).wait()


def _set_types():
    return [
        pltpu.VMEM((CB * K + L, D), jnp.float32),
        pltpu.VMEM((CB * KP, D), jnp.float32),
        pltpu.VMEM((CB, KP), jnp.int32),
        pltpu.VMEM((CB * K + L,), jnp.int32),
        pltpu.VMEM((CB * NC, D), jnp.float32),
        pltpu.VMEM((CB * NS,), jnp.float32),
        pltpu.VMEM((CB * NC,), jnp.float32),
        pltpu.VMEM((CB, D), jnp.float32),
        pltpu.VMEM((CB * K + L,), jnp.float32),
        pltpu.VMEM((CB * K + L,), jnp.int32),
    ]


@jax.jit
def _run(v_f, si_p, si_cf, ci_f, subr, sw_f, ctr_f, cw_f, user):
    mesh = plsc.VectorSubcoreMesh(core_axis_name="c", subcore_axis_name="s")
    f = pl.kernel(
        _body,
        out_type=jax.ShapeDtypeStruct((B * K,), jnp.float32),
        mesh=mesh,
        compiler_params=pltpu.CompilerParams(needs_layout_passes=False,
                                             use_tc_tiling_on_sc=False),
        scratch_types=(_set_types() + _set_types()
                       + [pltpu.SemaphoreType.DMA] * 8),
    )
    return f(v_f, si_p, si_cf, ci_f, subr, sw_f, ctr_f, cw_f, user)


def kernel(vectors, subcategory, category, subcategory_repr,
           subcategory_weights, category_repr, category_weights, user_repr):
    si32 = subcategory.astype(jnp.int32)
    si_p = jnp.pad(si32, ((0, 0), (0, KP - K)))
    si_cf = si32.reshape(B * K)
    v_f = vectors.reshape(B * K, D)
    ci_f = category.astype(jnp.int32).reshape(B * K)
    ctr_f = category_repr.reshape(B * NC, D)
    sw_f = subcategory_weights.reshape(B * NS)
    cw_f = category_weights.reshape(B * NC)
    out = _run(v_f, si_p, si_cf, ci_f, subcategory_repr, sw_f, ctr_f, cw_f,
               user_repr)
    return out.reshape(B, K)


# speedup vs baseline: 195.3726x; 1.0209x over previous
import jax
import jax.numpy as jnp
from jax import lax
from jax.experimental import pallas as pl
from jax.experimental.pallas import tpu as pltpu
from jax.experimental.pallas import tpu_sc as plsc

B, K, D, NS, NC = 4096, 50, 64, 256, 18
LAM_T, LAM_S = 0.15, 0.7
LAM_U = 1.0 - LAM_S - LAM_T
L = 16
NWORK = 32
BPW = B // NWORK
CB = 4
NCHUNK = BPW // CB
KG = 4
UNROLL = 16


def _lin_descs(hb, s, b0, sem):
    v_h, si_h, ci_h, sw_h, ctr_h, cw_h, u_h = hb
    v_l, es_l, si_c, ci_c, ct_l, sw_l, cw_l, u_l, out_l = s
    blk = pl.ds(b0, CB)
    return [
        (v_h.at[blk], v_l, sem),
        (si_h.at[blk], si_c, sem),
        (ci_h.at[blk], ci_c, sem),
        (ctr_h.at[blk], ct_l, sem),
        (sw_h.at[blk], sw_l, sem),
        (cw_h.at[blk], cw_l, sem),
        (u_h.at[blk], u_l, sem),
    ]


def _gather_descs(subr_h, s, si_s, b0, sem):
    es_l = s[1]
    return [
        (subr_h.at[b0 + j].at[si_s.at[j]], es_l.at[j], sem)
        for j in range(CB)
    ]


def _compute(s, iota):
    v_l, es_l, si_c, ci_c, ct_l, sw_l, cw_l, u_l, out_l = s
    for j in range(CB):
        jv = jnp.full((L,), j, jnp.int32)
        for g in range(KG):
            kvec = iota + g * L
            kvc = jnp.minimum(kvec, K - 1)
            civ = plsc.load_gather(ci_c, [jv, kvc])

            def dstep(it, acc, jv=jv, kvc=kvc, civ=civ, j=j):
                accs, accc, accu, cd = acc
                uvec = u_l[j, pl.ds(it * UNROLL, UNROLL)]
                for dd in range(UNROLL):
                    vg = plsc.load_gather(v_l, [jv, kvc, cd])
                    eg = plsc.load_gather(es_l, [jv, kvc, cd])
                    cg = plsc.load_gather(ct_l, [jv, civ, cd])
                    us = uvec[dd]
                    accs = accs + vg * eg
                    accc = accc + vg * cg
                    accu = accu + vg * us
                    cd = cd + 1
                return accs, accc, accu, cd

            zf = jnp.zeros((L,), jnp.float32)
            zi = jnp.zeros((L,), jnp.int32)
            accs, accc, accu, _ = lax.fori_loop(
                0, D // UNROLL, dstep, (zf, zf, zf, zi))

            siv = plsc.load_gather(si_c, [jv, kvc])
            ws = plsc.load_gather(sw_l, [jv, siv])
            wc = plsc.load_gather(cw_l, [jv, civ])
            outv = LAM_S * ws * accs + LAM_T * wc * accc + LAM_U * accu
            plsc.store_scatter(out_l, [jv, kvc], outv, mask=kvec < K)


def _body(v_h, si_h, ci_h, subr_h, sw_h, ctr_h, cw_h, u_h, out_h, *sc):
    sets = [sc[0:9], sc[9:18]]
    si_s = sc[18:20]
    sem_l = sc[20:22]
    sem_g = sc[22:24]
    sem_s = sc[24:26]
    sem_o = sc[26:28]
    hb = (v_h, si_h, ci_h, sw_h, ctr_h, cw_h, u_h)
    wid = lax.axis_index("s") * 2 + lax.axis_index("c")
    b0w = wid * BPW
    iota = lax.iota(jnp.int32, L)

    def si_copy(b0, j2):
        return (si_h.at[pl.ds(b0, CB)], si_s[j2], sem_s[j2])

    pltpu.async_copy(*si_copy(b0w, 0))
    pltpu.async_copy(*si_copy(b0w + CB, 1))
    pltpu.make_async_copy(*si_copy(b0w, 0)).wait()
    for d in _lin_descs(hb, sets[0], b0w, sem_l[0]):
        pltpu.async_copy(*d)
    for d in _gather_descs(subr_h, sets[0], si_s[0], b0w, sem_g[0]):
        pltpu.async_copy(*d)

    def step(sstep, carry):
        for j2 in range(2):
            c = 2 * sstep + j2
            b0 = b0w + c * CB
            s, o = sets[j2], sets[1 - j2]
            for d in _lin_descs(hb, s, b0, sem_l[j2]):
                pltpu.make_async_copy(*d).wait()
            for d in _gather_descs(subr_h, s, si_s[j2], b0, sem_g[j2]):
                pltpu.make_async_copy(*d).wait()

            @pl.when(c + 1 < NCHUNK)
            def _():
                bn = b0 + CB
                pltpu.make_async_copy(*si_copy(bn, 1 - j2)).wait()
                for d in _lin_descs(hb, o, bn, sem_l[1 - j2]):
                    pltpu.async_copy(*d)
                for d in _gather_descs(subr_h, o, si_s[1 - j2], bn,
                                       sem_g[1 - j2]):
                    pltpu.async_copy(*d)

            @pl.when(c + 2 < NCHUNK)
            def _():
                pltpu.async_copy(*si_copy(b0 + 2 * CB, j2))

            @pl.when(c >= 2)
            def _():
                pltpu.make_async_copy(
                    s[8], out_h.at[pl.ds(b0w, CB)], sem_o[j2]).wait()

            _compute(s, iota)
            pltpu.async_copy(s[8], out_h.at[pl.ds(b0, CB)], sem_o[j2])
        return carry

    lax.fori_loop(0, NCHUNK // 2, step, 0)
    for j2 in range(2):
        pltpu.make_async_copy(sets[j2][8], out_h.at[pl.ds(b0w, CB)],
                              sem_o[j2]).wait()


def _set_types():
    return [
        pltpu.VMEM((CB, K, D), jnp.float32),
        pltpu.VMEM((CB, K, D), jnp.float32),
        pltpu.VMEM((CB, K), jnp.int32),
        pltpu.VMEM((CB, K), jnp.int32),
        pltpu.VMEM((CB, NC, D), jnp.float32),
        pltpu.VMEM((CB, NS), jnp.float32),
        pltpu.VMEM((CB, NC), jnp.float32),
        pltpu.VMEM((CB, D), jnp.float32),
        pltpu.VMEM((CB, K), jnp.float32),
    ]


@jax.jit
def _run(v, si, ci, subr, sw, ctr, cw, user):
    mesh = plsc.VectorSubcoreMesh(core_axis_name="c", subcore_axis_name="s")
    f = pl.kernel(
        _body,
        out_type=jax.ShapeDtypeStruct((B, K), jnp.float32),
        mesh=mesh,
        compiler_params=pltpu.CompilerParams(needs_layout_passes=False,
                                             use_tc_tiling_on_sc=False),
        scratch_types=(_set_types() + _set_types()
                       + [pltpu.VMEM((CB, K), jnp.int32)] * 2
                       + [pltpu.SemaphoreType.DMA] * 8),
    )
    return f(v, si, ci, subr, sw, ctr, cw, user)


def kernel(vectors, subcategory, category, subcategory_repr,
           subcategory_weights, category_repr, category_weights, user_repr):
    return _run(vectors, subcategory, category, subcategory_repr,
                subcategory_weights, category_repr, category_weights,
                user_repr)


# speedup vs baseline: 305.1441x; 1.5619x over previous
import jax
import jax.numpy as jnp
from jax import lax
from jax.experimental import pallas as pl
from jax.experimental.pallas import tpu as pltpu
from jax.experimental.pallas import tpu_sc as plsc

B, K, D, NS, NC = 4096, 50, 64, 256, 18
LAM_T, LAM_S = 0.15, 0.7
LAM_U = 1.0 - LAM_S - LAM_T
L = 16
NWORK = 32
BPW = B // NWORK
CB = 4
NCHUNK = BPW // CB
KG = 4
UNROLL = 16


def _lin_descs(hb, s, b0, sem):
    v_h, si_h, ci_h, sw_h, ctr_h, cw_h, u_h = hb
    v_l, es_l, si_c, ci_c, ct_l, sw_l, cw_l, u_l, out_l = s
    blk = pl.ds(b0, CB)
    return [
        (v_h.at[blk], v_l, sem),
        (si_h.at[blk], si_c, sem),
        (ci_h.at[blk], ci_c, sem),
        (ctr_h.at[blk], ct_l, sem),
        (sw_h.at[blk], sw_l, sem),
        (cw_h.at[blk], cw_l, sem),
        (u_h.at[blk], u_l, sem),
    ]


def _gather_descs(subr_h, s, si_s, b0, sem):
    es_l = s[1]
    return [
        (subr_h.at[b0 + j].at[si_s.at[j]], es_l.at[j], sem)
        for j in range(CB)
    ]


def _compute(s, iota):
    v_l, es_l, si_c, ci_c, ct_l, sw_l, cw_l, u_l, out_l = s
    for j in range(CB):
        jv = jnp.full((L,), j, jnp.int32)
        for g in range(KG):
            kvec = iota + g * L
            kvc = jnp.minimum(kvec, K - 1)
            civ = plsc.load_gather(ci_c, [jv, kvc])

            def dstep(it, acc, jv=jv, kvc=kvc, civ=civ):
                accs, accc, accu, dv = acc
                for dd in range(UNROLL):
                    vg = plsc.load_gather(v_l, [jv, kvc, dv])
                    eg = plsc.load_gather(es_l, [jv, kvc, dv])
                    cg = plsc.load_gather(ct_l, [jv, civ, dv])
                    ug = plsc.load_gather(u_l, [jv, dv])
                    accs = accs + vg * eg
                    accc = accc + vg * cg
                    accu = accu + vg * ug
                    dv = (dv + 1) & (D - 1)
                return accs, accc, accu, dv

            zf = jnp.zeros((L,), jnp.float32)
            accs, accc, accu, _ = lax.fori_loop(
                0, D // UNROLL, dstep, (zf, zf, zf, iota))

            siv = plsc.load_gather(si_c, [jv, kvc])
            ws = plsc.load_gather(sw_l, [jv, siv])
            wc = plsc.load_gather(cw_l, [jv, civ])
            outv = LAM_S * ws * accs + LAM_T * wc * accc + LAM_U * accu
            plsc.store_scatter(out_l, [jv, kvc], outv, mask=kvec < K)


def _body(v_h, si_h, ci_h, subr_h, sw_h, ctr_h, cw_h, u_h, out_h, *sc):
    sets = [sc[0:9], sc[9:18]]
    si_s = sc[18:20]
    sem_l = sc[20:22]
    sem_g = sc[22:24]
    sem_s = sc[24:26]
    sem_o = sc[26:28]
    hb = (v_h, si_h, ci_h, sw_h, ctr_h, cw_h, u_h)
    wid = lax.axis_index("s") * 2 + lax.axis_index("c")
    b0w = wid * BPW
    iota = lax.iota(jnp.int32, L)

    def si_copy(b0, j2):
        return (si_h.at[pl.ds(b0, CB)], si_s[j2], sem_s[j2])

    pltpu.async_copy(*si_copy(b0w, 0))
    pltpu.async_copy(*si_copy(b0w + CB, 1))
    pltpu.make_async_copy(*si_copy(b0w, 0)).wait()
    for d in _lin_descs(hb, sets[0], b0w, sem_l[0]):
        pltpu.async_copy(*d)
    for d in _gather_descs(subr_h, sets[0], si_s[0], b0w, sem_g[0]):
        pltpu.async_copy(*d)

    def step(sstep, carry):
        for j2 in range(2):
            c = 2 * sstep + j2
            b0 = b0w + c * CB
            s, o = sets[j2], sets[1 - j2]
            for d in _lin_descs(hb, s, b0, sem_l[j2]):
                pltpu.make_async_copy(*d).wait()
            for d in _gather_descs(subr_h, s, si_s[j2], b0, sem_g[j2]):
                pltpu.make_async_copy(*d).wait()

            @pl.when(c + 1 < NCHUNK)
            def _():
                bn = b0 + CB
                pltpu.make_async_copy(*si_copy(bn, 1 - j2)).wait()
                for d in _lin_descs(hb, o, bn, sem_l[1 - j2]):
                    pltpu.async_copy(*d)
                for d in _gather_descs(subr_h, o, si_s[1 - j2], bn,
                                       sem_g[1 - j2]):
                    pltpu.async_copy(*d)

            @pl.when(c + 2 < NCHUNK)
            def _():
                pltpu.async_copy(*si_copy(b0 + 2 * CB, j2))

            @pl.when(c >= 2)
            def _():
                pltpu.make_async_copy(
                    s[8], out_h.at[pl.ds(b0w, CB)], sem_o[j2]).wait()

            _compute(s, iota)
            pltpu.async_copy(s[8], out_h.at[pl.ds(b0, CB)], sem_o[j2])
        return carry

    lax.fori_loop(0, NCHUNK // 2, step, 0)
    for j2 in range(2):
        pltpu.make_async_copy(sets[j2][8], out_h.at[pl.ds(b0w, CB)],
                              sem_o[j2]).wait()


def _set_types():
    return [
        pltpu.VMEM((CB, K, D), jnp.float32),
        pltpu.VMEM((CB, K, D), jnp.float32),
        pltpu.VMEM((CB, K), jnp.int32),
        pltpu.VMEM((CB, K), jnp.int32),
        pltpu.VMEM((CB, NC, D), jnp.float32),
        pltpu.VMEM((CB, NS), jnp.float32),
        pltpu.VMEM((CB, NC), jnp.float32),
        pltpu.VMEM((CB, D), jnp.float32),
        pltpu.VMEM((CB, K), jnp.float32),
    ]


@jax.jit
def _run(v, si, ci, subr, sw, ctr, cw, user):
    mesh = plsc.VectorSubcoreMesh(core_axis_name="c", subcore_axis_name="s")
    f = pl.kernel(
        _body,
        out_type=jax.ShapeDtypeStruct((B, K), jnp.float32),
        mesh=mesh,
        compiler_params=pltpu.CompilerParams(needs_layout_passes=False,
                                             use_tc_tiling_on_sc=False),
        scratch_types=(_set_types() + _set_types()
                       + [pltpu.VMEM((CB, K), jnp.int32)] * 2
                       + [pltpu.SemaphoreType.DMA] * 8),
    )
    return f(v, si, ci, subr, sw, ctr, cw, user)


def kernel(vectors, subcategory, category, subcategory_repr,
           subcategory_weights, category_repr, category_weights, user_repr):
    return _run(vectors, subcategory, category, subcategory_repr,
                subcategory_weights, category_repr, category_weights,
                user_repr)
